# baseline (speedup 1.0000x reference)
import functools

import jax
import jax.numpy as jnp
from jax import lax
from jax.experimental import pallas as pl
from jax.experimental.pallas import tpu as pltpu
import jax.experimental.pallas.tpu_sc as plsc

N = 10000
E = 320000
H = 16

NC = 2
NS = 16
NW = NC * NS
BE = 128
NCH = 78
XW = E // BE - NCH * NW
D = 6
LAG = 3
SPLIT = 40
ZR = 624
TAIL = N - ZR * NS
NPR = 1280
NN = NPR * 128 // H

_mesh = plsc.VectorSubcoreMesh(core_axis_name="c", subcore_axis_name="s")
_sc_params = pltpu.CompilerParams(use_tc_tiling_on_sc=False)


def _zero_rows(buf, nrows):
    zv = jnp.zeros((H,), jnp.float32)

    def body(i, _):
        buf[i] = zv
        return 0

    lax.fori_loop(0, nrows, body, 0)


def _zero_acc(acc, zbuf, sid):
    _zero_rows(zbuf, ZR)
    pltpu.sync_copy(zbuf, acc.at[pl.ds(sid * ZR, ZR)])

    @pl.when(sid == 0)
    def _():
        pltpu.sync_copy(zbuf.at[pl.ds(0, TAIL)], acc.at[pl.ds(ZR * NS, TAIL)])


def _copy_out(acc, out_hbm, cid, sid):
    pltpu.sync_copy(acc.at[pl.ds(sid * ZR, ZR)],
                    out_hbm.at[cid, pl.ds(sid * ZR, ZR)])

    @pl.when(sid == 0)
    def _():
        pltpu.sync_copy(acc.at[pl.ds(ZR * NS, TAIL)],
                        out_hbm.at[cid, pl.ds(ZR * NS, TAIL)])


def _load_chunk_indices(e_hbm, idx_v, wid):
    c0 = NCH * wid + jnp.minimum(wid, XW)
    pltpu.sync_copy(e_hbm.at[pl.ds(c0, NCH)], idx_v.at[pl.ds(0, NCH)])

    @pl.when(wid < XW)
    def _():
        pltpu.sync_copy(e_hbm.at[pl.ds(c0 + NCH, 1)], idx_v.at[pl.ds(NCH, 1)])


@functools.partial(
    pl.kernel,
    out_type=jax.ShapeDtypeStruct((NC, NN, H), jnp.float32),
    mesh=_mesh,
    compiler_params=_sc_params,
    scratch_types=[
        pltpu.VMEM((NCH + 1, BE), jnp.int32),
        pltpu.VMEM((BE, H), jnp.float32),
        pltpu.VMEM((ZR, H), jnp.float32),
        pltpu.VMEM_SHARED((N, H), jnp.float32),
        pltpu.SemaphoreType.DMA,
    ],
)
def _sc_degree(ed_hbm, out_hbm, didx, ones_v, zbuf, acc, sem):
    cid = lax.axis_index("c")
    sid = lax.axis_index("s")
    wid = cid * NS + sid

    ov = jnp.ones((H,), jnp.float32)

    def fill_ones(i, _):
        ones_v[i] = ov
        return 0

    lax.fori_loop(0, BE, fill_ones, 0)
    _zero_acc(acc, zbuf, sid)
    _load_chunk_indices(ed_hbm, didx, wid)
    plsc.subcore_barrier()

    def step(j, _):
        pltpu.async_copy(ones_v, acc.at[didx.at[j]], sem, add=True)

        @pl.when(j >= 8)
        def _():
            pltpu.make_async_copy(ones_v, acc.at[didx.at[0]], sem).wait()

        return 0

    lax.fori_loop(0, NCH, step, 0)

    def drain(j, _):
        pltpu.make_async_copy(ones_v, acc.at[didx.at[0]], sem).wait()
        return 0

    lax.fori_loop(0, 8, drain, 0)

    @pl.when(wid < XW)
    def _():
        pltpu.sync_copy(ones_v, acc.at[didx.at[NCH]], add=True)

    plsc.subcore_barrier()
    _copy_out(acc, out_hbm, cid, sid)


@functools.partial(
    pl.kernel,
    out_type=jax.ShapeDtypeStruct((NC, NN, H), jnp.float32),
    mesh=_mesh,
    compiler_params=_sc_params,
    scratch_types=[
        pltpu.VMEM((NCH + 1, BE), jnp.int32),
        pltpu.VMEM((NCH + 1, BE), jnp.int32),
        pltpu.VMEM((D, BE, H), jnp.float32),
        pltpu.VMEM((ZR, H), jnp.float32),
        pltpu.VMEM_SHARED((N, H), jnp.float32),
        pltpu.VMEM_SHARED((N, H), jnp.float32),
    ] + [pltpu.SemaphoreType.DMA] * (2 * D),
)
def _sc_aggregate(g_hbm, es_hbm, ed_hbm, out_hbm,
                  sidx, didx, gbuf, zbuf, acc, g_s, *sems):
    semg = sems[:D]
    sems_ = sems[D:]
    cid = lax.axis_index("c")
    sid = lax.axis_index("s")
    wid = cid * NS + sid

    _zero_acc(acc, zbuf, sid)
    pltpu.sync_copy(g_hbm.at[pl.ds(sid * ZR, ZR)], g_s.at[pl.ds(sid * ZR, ZR)])

    @pl.when(sid == 0)
    def _():
        pltpu.sync_copy(g_hbm.at[pl.ds(ZR * NS, TAIL)],
                        g_s.at[pl.ds(ZR * NS, TAIL)])

    _load_chunk_indices(es_hbm, sidx, wid)
    _load_chunk_indices(ed_hbm, didx, wid)
    plsc.subcore_barrier()

    for b in range(LAG):
        pltpu.async_copy(g_hbm.at[sidx.at[b]], gbuf.at[b], semg[b])

    def group(m, _):
        for b in range(D):
            j = m * D + b
            bg = (b + LAG) % D

            @pl.when(j >= LAG)
            def _():
                pltpu.make_async_copy(gbuf.at[bg], acc.at[didx.at[0]],
                                      sems_[bg]).wait()

            jj = j + LAG

            @pl.when(jj < SPLIT)
            def _():
                pltpu.async_copy(g_hbm.at[sidx.at[jj]], gbuf.at[bg],
                                 semg[bg])

            @pl.when((jj >= SPLIT) & (jj < NCH))
            def _():
                pltpu.async_copy(g_s.at[sidx.at[jj]], gbuf.at[bg],
                                 semg[bg])

            pltpu.make_async_copy(g_s.at[sidx.at[0]], gbuf.at[b],
                                  semg[b]).wait()
            pltpu.async_copy(gbuf.at[b], acc.at[didx.at[j]], sems_[b],
                             add=True)
        return 0

    lax.fori_loop(0, NCH // D, group, 0)
    for b in range(LAG, D):
        pltpu.make_async_copy(gbuf.at[b], acc.at[didx.at[0]], sems_[b]).wait()

    @pl.when(wid < XW)
    def _():
        pltpu.async_copy(g_s.at[sidx.at[NCH]], gbuf.at[0], semg[0])
        pltpu.make_async_copy(g_s.at[sidx.at[NCH]], gbuf.at[0],
                              semg[0]).wait()
        pltpu.sync_copy(gbuf.at[0], acc.at[didx.at[NCH]], add=True)

    plsc.subcore_barrier()
    _copy_out(acc, out_hbm, cid, sid)


NPK = N * H // 128


def _tc_h1(x_ref, w1_ref, h1_ref):
    w1 = w1_ref[...]
    cols = [
        jnp.dot(x_ref[:, i, :], w1, preferred_element_type=jnp.float32)
        for i in range(8)
    ]
    h1_ref[...] = jnp.concatenate(
        [jnp.concatenate(cols, axis=1),
         jnp.zeros((NPR - NPK, 128), jnp.float32)], axis=0)


def _tc_scale(degp_ref, h1_ref, dinv_ref, g1_ref):
    deg = degp_ref[0] + degp_ref[1] + 1.0
    dinv = lax.rsqrt(deg)
    dinv_ref[...] = dinv
    g1_ref[...] = dinv * h1_ref[...]


def _tc_mid(sp_ref, dinv_ref, h1_ref, b1_ref, w2k_ref, h2_ref, g2_ref):
    dinv = dinv_ref[...]
    s = sp_ref[0] + sp_ref[1]
    c1 = jnp.maximum(dinv * s + dinv * dinv * h1_ref[...] + b1_ref[...], 0.0)
    h2 = jnp.dot(c1, w2k_ref[...], preferred_element_type=jnp.float32)
    h2_ref[...] = h2
    g2_ref[...] = dinv * h2


def _tc_post(sp_ref, dinv_ref, h2_ref, b2_ref, wlin_ref, blin_ref, y_ref):
    dinv = dinv_ref[...]
    s = sp_ref[0] + sp_ref[1]
    c2p = jnp.maximum(dinv * s + dinv * dinv * h2_ref[...] + b2_ref[...], 0.0)
    wlin = wlin_ref[...]
    blin = blin_ref[...]
    for i in range(8):
        ci = c2p[:NPK, i * H:(i + 1) * H]
        y_ref[:, i, :] = (
            jnp.dot(ci, wlin, preferred_element_type=jnp.float32) + blin
        )


def kernel(x, edge_index, W1, b1, W2, b2, Wlin, blin):
    es = edge_index[0].reshape(E // BE, BE)
    ed = edge_index[1].reshape(E // BE, BE)
    eye8 = jnp.eye(8, dtype=jnp.float32)
    w2k = jnp.kron(eye8, W2)
    b1t = jnp.tile(b1, 8).reshape(1, 128)
    b2t = jnp.tile(b2, 8).reshape(1, 128)
    blinr = blin.reshape(1, -1)
    f32 = jnp.float32
    packed = jax.ShapeDtypeStruct((NPR, 128), f32)

    degp = _sc_degree(ed).reshape(NC, NPR, 128)

    h1 = pl.pallas_call(
        _tc_h1, out_shape=packed,
    )(x.reshape(NPK, 8, 128), W1)

    dinv, g1 = pl.pallas_call(
        _tc_scale, out_shape=[packed] * 2,
    )(degp, h1)

    s1p = _sc_aggregate(g1.reshape(NN, H), es, ed).reshape(NC, NPR, 128)

    h2, g2 = pl.pallas_call(
        _tc_mid, out_shape=[packed] * 2,
    )(s1p, dinv, h1, b1t, w2k)

    s2p = _sc_aggregate(g2.reshape(NN, H), es, ed).reshape(NC, NPR, 128)

    y = pl.pallas_call(
        _tc_post, out_shape=jax.ShapeDtypeStruct((NPK, 8, 128), f32),
    )(s2p, dinv, h2, b2t, Wlin, blinr)
    return y.reshape(N, x.shape[1])


# speedup vs baseline: 1.0563x; 1.0563x over previous
import functools

import jax
import jax.numpy as jnp
from jax import lax
from jax.experimental import pallas as pl
from jax.experimental.pallas import tpu as pltpu
import jax.experimental.pallas.tpu_sc as plsc

N = 10000
E = 320000
H = 16

NC = 2
NS = 16
NW = NC * NS
BE = 128
NCH = 78
XW = E // BE - NCH * NW
D = 6
LAG = 3
SPLIT = 40
ZR = 624
TAIL = N - ZR * NS
NPR = 1280
NN = NPR * 128 // H

_mesh = plsc.VectorSubcoreMesh(core_axis_name="c", subcore_axis_name="s")
_sc_params = pltpu.CompilerParams(use_tc_tiling_on_sc=False)


def _zero_rows(buf, nrows):
    zv = jnp.zeros((H,), jnp.float32)

    def body(i, _):
        buf[i] = zv
        return 0

    lax.fori_loop(0, nrows, body, 0)


def _zero_acc(acc, zbuf, sid):
    _zero_rows(zbuf, ZR)
    pltpu.sync_copy(zbuf, acc.at[pl.ds(sid * ZR, ZR)])

    @pl.when(sid == 0)
    def _():
        pltpu.sync_copy(zbuf.at[pl.ds(0, TAIL)], acc.at[pl.ds(ZR * NS, TAIL)])


def _copy_out(acc, out_hbm, cid, sid):
    pltpu.sync_copy(acc.at[pl.ds(sid * ZR, ZR)],
                    out_hbm.at[cid, pl.ds(sid * ZR, ZR)])

    @pl.when(sid == 0)
    def _():
        pltpu.sync_copy(acc.at[pl.ds(ZR * NS, TAIL)],
                        out_hbm.at[cid, pl.ds(ZR * NS, TAIL)])


def _load_chunk_indices(e_hbm, idx_v, wid):
    c0 = NCH * wid + jnp.minimum(wid, XW)
    pltpu.sync_copy(e_hbm.at[pl.ds(c0, NCH)], idx_v.at[pl.ds(0, NCH)])

    @pl.when(wid < XW)
    def _():
        pltpu.sync_copy(e_hbm.at[pl.ds(c0 + NCH, 1)], idx_v.at[pl.ds(NCH, 1)])


@functools.partial(
    pl.kernel,
    out_type=jax.ShapeDtypeStruct((NC, NN, H), jnp.float32),
    mesh=_mesh,
    compiler_params=_sc_params,
    scratch_types=[
        pltpu.VMEM((NCH + 1, BE), jnp.int32),
        pltpu.VMEM((BE, H), jnp.float32),
        pltpu.VMEM((ZR, H), jnp.float32),
        pltpu.VMEM_SHARED((N, H), jnp.float32),
        pltpu.SemaphoreType.DMA,
    ],
)
def _sc_degree(ed_hbm, out_hbm, didx, ones_v, zbuf, acc, sem):
    cid = lax.axis_index("c")
    sid = lax.axis_index("s")
    wid = cid * NS + sid

    ov = jnp.ones((H,), jnp.float32)

    def fill_ones(i, _):
        ones_v[i] = ov
        return 0

    lax.fori_loop(0, BE, fill_ones, 0)
    _zero_acc(acc, zbuf, sid)
    _load_chunk_indices(ed_hbm, didx, wid)
    plsc.subcore_barrier()

    def step(j, _):
        pltpu.async_copy(ones_v, acc.at[didx.at[j]], sem, add=True)

        @pl.when(j >= 8)
        def _():
            pltpu.make_async_copy(ones_v, acc.at[didx.at[0]], sem).wait()

        return 0

    lax.fori_loop(0, NCH, step, 0)

    def drain(j, _):
        pltpu.make_async_copy(ones_v, acc.at[didx.at[0]], sem).wait()
        return 0

    lax.fori_loop(0, 8, drain, 0)

    @pl.when(wid < XW)
    def _():
        pltpu.sync_copy(ones_v, acc.at[didx.at[NCH]], add=True)

    plsc.subcore_barrier()
    _copy_out(acc, out_hbm, cid, sid)


@functools.partial(
    pl.kernel,
    out_type=jax.ShapeDtypeStruct((NC, NN, H), jnp.float32),
    mesh=_mesh,
    compiler_params=_sc_params,
    scratch_types=[
        pltpu.VMEM((NCH + 1, BE), jnp.int32),
        pltpu.VMEM((NCH + 1, BE), jnp.int32),
        pltpu.VMEM((D, BE, H), jnp.float32),
        pltpu.VMEM((ZR, H), jnp.float32),
        pltpu.VMEM_SHARED((N, H), jnp.float32),
        pltpu.VMEM_SHARED((N, H), jnp.float32),
    ] + [pltpu.SemaphoreType.DMA] * (2 * D),
)
def _sc_aggregate(g_hbm, es_hbm, ed_hbm, out_hbm,
                  sidx, didx, gbuf, zbuf, acc, g_s, *sems):
    semg = sems[:D]
    sems_ = sems[D:]
    cid = lax.axis_index("c")
    sid = lax.axis_index("s")
    wid = cid * NS + sid

    _zero_acc(acc, zbuf, sid)
    pltpu.sync_copy(g_hbm.at[pl.ds(sid * ZR, ZR)], g_s.at[pl.ds(sid * ZR, ZR)])

    @pl.when(sid == 0)
    def _():
        pltpu.sync_copy(g_hbm.at[pl.ds(ZR * NS, TAIL)],
                        g_s.at[pl.ds(ZR * NS, TAIL)])

    _load_chunk_indices(es_hbm, sidx, wid)
    _load_chunk_indices(ed_hbm, didx, wid)
    plsc.subcore_barrier()

    for b in range(LAG):
        pltpu.async_copy(g_s.at[sidx.at[b]], gbuf.at[b], semg[b])

    def group(m, _):
        for b in range(D):
            j = m * D + b
            bg = (b + LAG) % D

            @pl.when(j >= LAG)
            def _():
                pltpu.make_async_copy(gbuf.at[bg], acc.at[didx.at[0]],
                                      sems_[bg]).wait()

            @pl.when(j + LAG < NCH)
            def _():
                pltpu.async_copy(g_s.at[sidx.at[j + LAG]], gbuf.at[bg],
                                 semg[bg])

            pltpu.make_async_copy(g_s.at[sidx.at[0]], gbuf.at[b],
                                  semg[b]).wait()
            pltpu.async_copy(gbuf.at[b], acc.at[didx.at[j]], sems_[b],
                             add=True)
        return 0

    lax.fori_loop(0, NCH // D, group, 0)
    for b in range(LAG, D):
        pltpu.make_async_copy(gbuf.at[b], acc.at[didx.at[0]], sems_[b]).wait()

    @pl.when(wid < XW)
    def _():
        pltpu.async_copy(g_s.at[sidx.at[NCH]], gbuf.at[0], semg[0])
        pltpu.make_async_copy(g_s.at[sidx.at[NCH]], gbuf.at[0],
                              semg[0]).wait()
        pltpu.sync_copy(gbuf.at[0], acc.at[didx.at[NCH]], add=True)

    plsc.subcore_barrier()
    _copy_out(acc, out_hbm, cid, sid)


NPK = N * H // 128


def _tc_h1(x_ref, w1_ref, h1_ref):
    w1 = w1_ref[...]
    cols = [
        jnp.dot(x_ref[:, i, :], w1, preferred_element_type=jnp.float32)
        for i in range(8)
    ]
    h1_ref[...] = jnp.concatenate(
        [jnp.concatenate(cols, axis=1),
         jnp.zeros((NPR - NPK, 128), jnp.float32)], axis=0)


def _tc_scale(degp_ref, h1_ref, dinv_ref, g1_ref):
    deg = degp_ref[0] + degp_ref[1] + 1.0
    dinv = lax.rsqrt(deg)
    dinv_ref[...] = dinv
    g1_ref[...] = dinv * h1_ref[...]


def _tc_mid(sp_ref, dinv_ref, h1_ref, b1_ref, w2k_ref, h2_ref, g2_ref):
    dinv = dinv_ref[...]
    s = sp_ref[0] + sp_ref[1]
    c1 = jnp.maximum(dinv * s + dinv * dinv * h1_ref[...] + b1_ref[...], 0.0)
    h2 = jnp.dot(c1, w2k_ref[...], preferred_element_type=jnp.float32)
    h2_ref[...] = h2
    g2_ref[...] = dinv * h2


def _tc_post(sp_ref, dinv_ref, h2_ref, b2_ref, wlin_ref, blin_ref, y_ref):
    dinv = dinv_ref[...]
    s = sp_ref[0] + sp_ref[1]
    c2p = jnp.maximum(dinv * s + dinv * dinv * h2_ref[...] + b2_ref[...], 0.0)
    wlin = wlin_ref[...]
    blin = blin_ref[...]
    for i in range(8):
        ci = c2p[:NPK, i * H:(i + 1) * H]
        y_ref[:, i, :] = (
            jnp.dot(ci, wlin, preferred_element_type=jnp.float32) + blin
        )


def kernel(x, edge_index, W1, b1, W2, b2, Wlin, blin):
    es = edge_index[0].reshape(E // BE, BE)
    ed = edge_index[1].reshape(E // BE, BE)
    eye8 = jnp.eye(8, dtype=jnp.float32)
    w2k = jnp.kron(eye8, W2)
    b1t = jnp.tile(b1, 8).reshape(1, 128)
    b2t = jnp.tile(b2, 8).reshape(1, 128)
    blinr = blin.reshape(1, -1)
    f32 = jnp.float32
    packed = jax.ShapeDtypeStruct((NPR, 128), f32)

    degp = _sc_degree(ed).reshape(NC, NPR, 128)

    h1 = pl.pallas_call(
        _tc_h1, out_shape=packed,
    )(x.reshape(NPK, 8, 128), W1)

    dinv, g1 = pl.pallas_call(
        _tc_scale, out_shape=[packed] * 2,
    )(degp, h1)

    s1p = _sc_aggregate(g1.reshape(NN, H), es, ed).reshape(NC, NPR, 128)

    h2, g2 = pl.pallas_call(
        _tc_mid, out_shape=[packed] * 2,
    )(s1p, dinv, h1, b1t, w2k)

    s2p = _sc_aggregate(g2.reshape(NN, H), es, ed).reshape(NC, NPR, 128)

    y = pl.pallas_call(
        _tc_post, out_shape=jax.ShapeDtypeStruct((NPK, 8, 128), f32),
    )(s2p, dinv, h2, b2t, Wlin, blinr)
    return y.reshape(N, x.shape[1])


# speedup vs baseline: 1.0566x; 1.0003x over previous
import functools

import jax
import jax.numpy as jnp
from jax import lax
from jax.experimental import pallas as pl
from jax.experimental.pallas import tpu as pltpu
import jax.experimental.pallas.tpu_sc as plsc

N = 10000
E = 320000
H = 16

NC = 2
NS = 16
NW = NC * NS
BE = 128
NCH = 78
XW = E // BE - NCH * NW
D = 6
LAG = 3
SPLIT = 40
ZR = 624
TAIL = N - ZR * NS
NPR = 1280
NN = NPR * 128 // H

_mesh = plsc.VectorSubcoreMesh(core_axis_name="c", subcore_axis_name="s")
_sc_params = pltpu.CompilerParams(use_tc_tiling_on_sc=False)


def _zero_rows(buf, nrows):
    zv = jnp.zeros((H,), jnp.float32)

    def body(i, _):
        buf[i] = zv
        return 0

    lax.fori_loop(0, nrows, body, 0)


def _zero_acc(acc, zbuf, sid):
    _zero_rows(zbuf, ZR)
    pltpu.sync_copy(zbuf, acc.at[pl.ds(sid * ZR, ZR)])

    @pl.when(sid == 0)
    def _():
        pltpu.sync_copy(zbuf.at[pl.ds(0, TAIL)], acc.at[pl.ds(ZR * NS, TAIL)])


def _copy_out(acc, out_hbm, cid, sid):
    pltpu.sync_copy(acc.at[pl.ds(sid * ZR, ZR)],
                    out_hbm.at[cid, pl.ds(sid * ZR, ZR)])

    @pl.when(sid == 0)
    def _():
        pltpu.sync_copy(acc.at[pl.ds(ZR * NS, TAIL)],
                        out_hbm.at[cid, pl.ds(ZR * NS, TAIL)])


def _load_chunk_indices(e_hbm, idx_v, wid):
    c0 = NCH * wid + jnp.minimum(wid, XW)
    pltpu.sync_copy(e_hbm.at[pl.ds(c0, NCH)], idx_v.at[pl.ds(0, NCH)])

    @pl.when(wid < XW)
    def _():
        pltpu.sync_copy(e_hbm.at[pl.ds(c0 + NCH, 1)], idx_v.at[pl.ds(NCH, 1)])


@functools.partial(
    pl.kernel,
    out_type=jax.ShapeDtypeStruct((NC, NN, H), jnp.float32),
    mesh=_mesh,
    compiler_params=_sc_params,
    scratch_types=[
        pltpu.VMEM((NCH + 1, BE), jnp.int32),
        pltpu.VMEM((BE, H), jnp.float32),
        pltpu.VMEM((ZR, H), jnp.float32),
        pltpu.VMEM_SHARED((N, H), jnp.float32),
        pltpu.SemaphoreType.DMA,
    ],
)
def _sc_degree(ed_hbm, out_hbm, didx, ones_v, zbuf, acc, sem):
    cid = lax.axis_index("c")
    sid = lax.axis_index("s")
    wid = cid * NS + sid

    ov = jnp.ones((H,), jnp.float32)

    def fill_ones(i, _):
        ones_v[i] = ov
        return 0

    lax.fori_loop(0, BE, fill_ones, 0)
    _zero_acc(acc, zbuf, sid)
    _load_chunk_indices(ed_hbm, didx, wid)
    plsc.subcore_barrier()

    def step(j, _):
        pltpu.async_copy(ones_v, acc.at[didx.at[j]], sem, add=True)

        @pl.when(j >= 16)
        def _():
            pltpu.make_async_copy(ones_v, acc.at[didx.at[0]], sem).wait()

        return 0

    lax.fori_loop(0, NCH, step, 0)

    def drain(j, _):
        pltpu.make_async_copy(ones_v, acc.at[didx.at[0]], sem).wait()
        return 0

    lax.fori_loop(0, 16, drain, 0)

    @pl.when(wid < XW)
    def _():
        pltpu.sync_copy(ones_v, acc.at[didx.at[NCH]], add=True)

    plsc.subcore_barrier()
    _copy_out(acc, out_hbm, cid, sid)


@functools.partial(
    pl.kernel,
    out_type=jax.ShapeDtypeStruct((NC, NN, H), jnp.float32),
    mesh=_mesh,
    compiler_params=_sc_params,
    scratch_types=[
        pltpu.VMEM((NCH + 1, BE), jnp.int32),
        pltpu.VMEM((NCH + 1, BE), jnp.int32),
        pltpu.VMEM((D, BE, H), jnp.float32),
        pltpu.VMEM((ZR, H), jnp.float32),
        pltpu.VMEM_SHARED((N, H), jnp.float32),
        pltpu.VMEM_SHARED((N, H), jnp.float32),
    ] + [pltpu.SemaphoreType.DMA] * (2 * D),
)
def _sc_aggregate(g_hbm, es_hbm, ed_hbm, out_hbm,
                  sidx, didx, gbuf, zbuf, acc, g_s, *sems):
    semg = sems[:D]
    sems_ = sems[D:]
    cid = lax.axis_index("c")
    sid = lax.axis_index("s")
    wid = cid * NS + sid

    _zero_acc(acc, zbuf, sid)
    pltpu.sync_copy(g_hbm.at[pl.ds(sid * ZR, ZR)], g_s.at[pl.ds(sid * ZR, ZR)])

    @pl.when(sid == 0)
    def _():
        pltpu.sync_copy(g_hbm.at[pl.ds(ZR * NS, TAIL)],
                        g_s.at[pl.ds(ZR * NS, TAIL)])

    _load_chunk_indices(es_hbm, sidx, wid)
    _load_chunk_indices(ed_hbm, didx, wid)
    plsc.subcore_barrier()

    for b in range(LAG):
        pltpu.async_copy(g_s.at[sidx.at[b]], gbuf.at[b], semg[b])

    def group(m, _):
        for b in range(D):
            j = m * D + b
            bg = (b + LAG) % D

            @pl.when(j >= LAG)
            def _():
                pltpu.make_async_copy(gbuf.at[bg], acc.at[didx.at[0]],
                                      sems_[bg]).wait()

            @pl.when(j + LAG < NCH)
            def _():
                pltpu.async_copy(g_s.at[sidx.at[j + LAG]], gbuf.at[bg],
                                 semg[bg])

            pltpu.make_async_copy(g_s.at[sidx.at[0]], gbuf.at[b],
                                  semg[b]).wait()
            pltpu.async_copy(gbuf.at[b], acc.at[didx.at[j]], sems_[b],
                             add=True)
        return 0

    lax.fori_loop(0, NCH // D, group, 0)
    for b in range(LAG, D):
        pltpu.make_async_copy(gbuf.at[b], acc.at[didx.at[0]], sems_[b]).wait()

    @pl.when(wid < XW)
    def _():
        pltpu.async_copy(g_s.at[sidx.at[NCH]], gbuf.at[0], semg[0])
        pltpu.make_async_copy(g_s.at[sidx.at[NCH]], gbuf.at[0],
                              semg[0]).wait()
        pltpu.sync_copy(gbuf.at[0], acc.at[didx.at[NCH]], add=True)

    plsc.subcore_barrier()
    _copy_out(acc, out_hbm, cid, sid)


NPK = N * H // 128


def _tc_h1(x_ref, w1_ref, h1_ref):
    w1 = w1_ref[...]
    cols = [
        jnp.dot(x_ref[:, i, :], w1, preferred_element_type=jnp.float32)
        for i in range(8)
    ]
    h1_ref[...] = jnp.concatenate(
        [jnp.concatenate(cols, axis=1),
         jnp.zeros((NPR - NPK, 128), jnp.float32)], axis=0)


def _tc_scale(degp_ref, h1_ref, dinv_ref, g1_ref):
    deg = degp_ref[0] + degp_ref[1] + 1.0
    dinv = lax.rsqrt(deg)
    dinv_ref[...] = dinv
    g1_ref[...] = dinv * h1_ref[...]


def _tc_mid(sp_ref, dinv_ref, h1_ref, b1_ref, w2k_ref, h2_ref, g2_ref):
    dinv = dinv_ref[...]
    s = sp_ref[0] + sp_ref[1]
    c1 = jnp.maximum(dinv * s + dinv * dinv * h1_ref[...] + b1_ref[...], 0.0)
    h2 = jnp.dot(c1, w2k_ref[...], preferred_element_type=jnp.float32)
    h2_ref[...] = h2
    g2_ref[...] = dinv * h2


def _tc_post(sp_ref, dinv_ref, h2_ref, b2_ref, wlin_ref, blin_ref, y_ref):
    dinv = dinv_ref[...]
    s = sp_ref[0] + sp_ref[1]
    c2p = jnp.maximum(dinv * s + dinv * dinv * h2_ref[...] + b2_ref[...], 0.0)
    wlin = wlin_ref[...]
    blin = blin_ref[...]
    for i in range(8):
        ci = c2p[:NPK, i * H:(i + 1) * H]
        y_ref[:, i, :] = (
            jnp.dot(ci, wlin, preferred_element_type=jnp.float32) + blin
        )


def kernel(x, edge_index, W1, b1, W2, b2, Wlin, blin):
    es = edge_index[0].reshape(E // BE, BE)
    ed = edge_index[1].reshape(E // BE, BE)
    eye8 = jnp.eye(8, dtype=jnp.float32)
    w2k = jnp.kron(eye8, W2)
    b1t = jnp.tile(b1, 8).reshape(1, 128)
    b2t = jnp.tile(b2, 8).reshape(1, 128)
    blinr = blin.reshape(1, -1)
    f32 = jnp.float32
    packed = jax.ShapeDtypeStruct((NPR, 128), f32)

    degp = _sc_degree(ed).reshape(NC, NPR, 128)

    h1 = pl.pallas_call(
        _tc_h1, out_shape=packed,
    )(x.reshape(NPK, 8, 128), W1)

    dinv, g1 = pl.pallas_call(
        _tc_scale, out_shape=[packed] * 2,
    )(degp, h1)

    s1p = _sc_aggregate(g1.reshape(NN, H), es, ed).reshape(NC, NPR, 128)

    h2, g2 = pl.pallas_call(
        _tc_mid, out_shape=[packed] * 2,
    )(s1p, dinv, h1, b1t, w2k)

    s2p = _sc_aggregate(g2.reshape(NN, H), es, ed).reshape(NC, NPR, 128)

    y = pl.pallas_call(
        _tc_post, out_shape=jax.ShapeDtypeStruct((NPK, 8, 128), f32),
    )(s2p, dinv, h2, b2t, Wlin, blinr)
    return y.reshape(N, x.shape[1])
